# baseline (speedup 1.0000x reference)
import jax
import jax.numpy as jnp
from jax import lax
from jax.experimental import pallas as pl
from jax.experimental.pallas import tpu as pltpu


def _leaky(x, slope=0.01):
    return jnp.where(x > 0, x, slope * x)


def _mlp_kernel(z_ref, w1_ref, b1_ref, w2_ref, b2_ref, t_ref,
                w1s_ref, b1s_ref, w2s_ref, b2s_ref):
    j = pl.program_id(1)
    r_count, _, h1 = w1_ref.shape
    h2 = w2_ref.shape[2]
    hp = t_ref.shape[1]

    @pl.when(j == 0)
    def _():
        zeros_col = jnp.zeros((z_ref.shape[1], hp - r_count * h1), jnp.float32)
        w1s_ref[...] = jnp.concatenate(
            [w1_ref[r] for r in range(r_count)] + [zeros_col], axis=1)
        b1s_ref[...] = jnp.concatenate(
            [b1_ref[r] for r in range(r_count)]
            + [jnp.zeros((1, hp - r_count * h1), jnp.float32)], axis=1)
        w2_rows = [
            jnp.concatenate(
                ([jnp.zeros((h1, r * h2), jnp.float32)] if r > 0 else [])
                + [w2_ref[r], jnp.zeros((h1, hp - (r + 1) * h2), jnp.float32)],
                axis=1)
            for r in range(r_count)
        ]
        w2s_ref[...] = jnp.concatenate(
            w2_rows + [jnp.zeros((hp - r_count * h1, hp), jnp.float32)], axis=0)
        b2s_ref[...] = jnp.concatenate(
            [b2_ref[r] for r in range(r_count)]
            + [jnp.zeros((1, hp - r_count * h2), jnp.float32)], axis=1)

    h = _leaky(jnp.dot(z_ref[...], w1s_ref[...],
                       preferred_element_type=jnp.float32) + b1s_ref[...])
    t = _leaky(jnp.dot(h, w2s_ref[...],
                       preferred_element_type=jnp.float32) + b2s_ref[...])
    t_ref[...] = t.astype(jnp.bfloat16)


def _gram_kernel(t_ref, out_ref):
    i = pl.program_id(0)
    tm = out_ref.shape[0]
    out_ref[...] = lax.dot_general(
        t_ref[pl.ds(i * tm, tm), :], t_ref[...],
        dimension_numbers=(((1,), (1,)), ((), ())),
        preferred_element_type=jnp.float32)


def kernel(z, w1, b1, w2, b2):
    z = z.astype(jnp.float32)
    w1 = w1.astype(jnp.float32)
    b1 = b1.astype(jnp.float32)
    w2 = w2.astype(jnp.float32)
    b2 = b2.astype(jnp.float32)
    N, D = z.shape
    R, _, H1 = w1.shape
    H2 = w2.shape[2]
    HP = 128

    TM1 = 1024
    NC = 2
    SPC1 = N // TM1 // NC
    t_mat = pl.pallas_call(
        _mlp_kernel,
        out_shape=jax.ShapeDtypeStruct((N, HP), jnp.bfloat16),
        grid=(NC, SPC1),
        in_specs=[
            pl.BlockSpec((TM1, D), lambda i, j: (i * SPC1 + j, 0)),
            pl.BlockSpec((R, D, H1), lambda i, j: (0, 0, 0)),
            pl.BlockSpec((R, 1, H1), lambda i, j: (0, 0, 0)),
            pl.BlockSpec((R, H1, H2), lambda i, j: (0, 0, 0)),
            pl.BlockSpec((R, 1, H2), lambda i, j: (0, 0, 0)),
        ],
        out_specs=pl.BlockSpec((TM1, HP), lambda i, j: (i * SPC1 + j, 0)),
        scratch_shapes=[
            pltpu.VMEM((D, HP), jnp.float32),
            pltpu.VMEM((1, HP), jnp.float32),
            pltpu.VMEM((HP, HP), jnp.float32),
            pltpu.VMEM((1, HP), jnp.float32),
        ],
        compiler_params=pltpu.CompilerParams(
            dimension_semantics=("parallel", "arbitrary")),
    )(z, w1, b1, w2, b2)

    TM = 256
    out = pl.pallas_call(
        _gram_kernel,
        out_shape=jax.ShapeDtypeStruct((N, N), jnp.float32),
        grid=(N // TM,),
        in_specs=[
            pl.BlockSpec((N, HP), lambda i: (0, 0)),
        ],
        out_specs=pl.BlockSpec((TM, N), lambda i: (i, 0)),
        compiler_params=pltpu.CompilerParams(
            dimension_semantics=("parallel",)),
        cost_estimate=pl.CostEstimate(
            flops=2 * N * N * HP, transcendentals=0,
            bytes_accessed=4 * N * N + 2 * 2 * N * HP),
    )(t_mat)
    return out


# speedup vs baseline: 1.0233x; 1.0233x over previous
import jax
import jax.numpy as jnp
from jax import lax
from jax.experimental import pallas as pl
from jax.experimental.pallas import tpu as pltpu


def _leaky(x, slope=0.01):
    return jnp.where(x > 0, x, slope * x)


def _mlp_kernel(z_ref, w1_ref, b1_ref, w2_ref, b2_ref, t_ref):
    r_count, _, h1 = w1_ref.shape
    h2 = w2_ref.shape[2]
    hp = t_ref.shape[1]
    zeros_col = jnp.zeros((z_ref.shape[1], hp - r_count * h1), jnp.float32)
    w1p = jnp.concatenate([w1_ref[r] for r in range(r_count)] + [zeros_col], axis=1)
    b1p = jnp.concatenate([b1_ref[r] for r in range(r_count)]
                          + [jnp.zeros((1, hp - r_count * h1), jnp.float32)], axis=1)
    w2_rows = [
        jnp.concatenate(
            ([jnp.zeros((h1, r * h2), jnp.float32)] if r > 0 else [])
            + [w2_ref[r], jnp.zeros((h1, hp - (r + 1) * h2), jnp.float32)],
            axis=1)
        for r in range(r_count)
    ]
    w2p = jnp.concatenate(
        w2_rows + [jnp.zeros((hp - r_count * h1, hp), jnp.float32)], axis=0)
    b2p = jnp.concatenate([b2_ref[r] for r in range(r_count)]
                          + [jnp.zeros((1, hp - r_count * h2), jnp.float32)], axis=1)

    h = _leaky(jnp.dot(z_ref[...], w1p, preferred_element_type=jnp.float32) + b1p)
    t = _leaky(jnp.dot(h, w2p, preferred_element_type=jnp.float32) + b2p)
    t_ref[...] = t.astype(jnp.bfloat16)


def _gram_kernel(t_ref, out_ref):
    i = pl.program_id(0)
    tm = out_ref.shape[0]
    out_ref[...] = lax.dot_general(
        t_ref[pl.ds(i * tm, tm), :], t_ref[...],
        dimension_numbers=(((1,), (1,)), ((), ())),
        preferred_element_type=jnp.float32)


def kernel(z, w1, b1, w2, b2):
    z = z.astype(jnp.float32)
    w1 = w1.astype(jnp.float32)
    b1 = b1.astype(jnp.float32)
    w2 = w2.astype(jnp.float32)
    b2 = b2.astype(jnp.float32)
    N, D = z.shape
    R, _, H1 = w1.shape
    H2 = w2.shape[2]
    HP = 128

    TM1 = 2048
    t_mat = pl.pallas_call(
        _mlp_kernel,
        out_shape=jax.ShapeDtypeStruct((N, HP), jnp.bfloat16),
        grid=(N // TM1,),
        in_specs=[
            pl.BlockSpec((TM1, D), lambda i: (i, 0)),
            pl.BlockSpec((R, D, H1), lambda i: (0, 0, 0)),
            pl.BlockSpec((R, 1, H1), lambda i: (0, 0, 0)),
            pl.BlockSpec((R, H1, H2), lambda i: (0, 0, 0)),
            pl.BlockSpec((R, 1, H2), lambda i: (0, 0, 0)),
        ],
        out_specs=pl.BlockSpec((TM1, HP), lambda i: (i, 0)),
        compiler_params=pltpu.CompilerParams(dimension_semantics=("parallel",)),
    )(z, w1, b1, w2, b2)

    TM = 256
    out = pl.pallas_call(
        _gram_kernel,
        out_shape=jax.ShapeDtypeStruct((N, N), jnp.float32),
        grid=(N // TM,),
        in_specs=[
            pl.BlockSpec((N, HP), lambda i: (0, 0)),
        ],
        out_specs=pl.BlockSpec((TM, N), lambda i: (i, 0)),
        compiler_params=pltpu.CompilerParams(
            dimension_semantics=("parallel",)),
        cost_estimate=pl.CostEstimate(
            flops=2 * N * N * HP, transcendentals=0,
            bytes_accessed=4 * N * N + 2 * 2 * N * HP),
    )(t_mat)
    return out
